# baseline (speedup 1.0000x reference)
import functools

import jax
import jax.numpy as jnp
from jax import lax
from jax.experimental import pallas as pl
from jax.experimental.pallas import tpu as pltpu
from jax.experimental.pallas import tpu_sc as plsc

NUM_CORES = 2
NUM_SUBCORES = 16
NUM_WORKERS = NUM_CORES * NUM_SUBCORES
LANES = 16


def _sc_add(B, N, D, p_hbm, t_hbm, o_hbm, tbl_v, buf_v):
    rows_per_worker = N // NUM_WORKERS
    chunk = rows_per_worker * D
    wid = lax.axis_index("s") * NUM_CORES + lax.axis_index("c")
    n0 = wid * chunk
    pltpu.sync_copy(t_hbm.at[pl.ds(n0, chunk)], tbl_v)

    def batch_body(b, carry):
        off = b * (N * D) + n0
        pltpu.sync_copy(p_hbm.at[pl.ds(off, chunk)], buf_v)

        def add_body(i, carry2):
            o = i * (4 * LANES)
            for u in range(4):
                s = pl.ds(o + u * LANES, LANES)
                buf_v[s] = buf_v[s] + tbl_v[s]
            return carry2

        lax.fori_loop(0, chunk // (4 * LANES), add_body, 0, unroll=2)
        pltpu.sync_copy(buf_v, o_hbm.at[pl.ds(off, chunk)])
        return carry

    lax.fori_loop(0, B, batch_body, 0)


def kernel(patches, pos_table):
    B, N, D = patches.shape
    table_flat = pos_table[:N].reshape(-1)
    patches_flat = patches.reshape(-1)
    rows_per_worker = N // NUM_WORKERS
    chunk = rows_per_worker * D

    mesh = plsc.VectorSubcoreMesh(core_axis_name="c", subcore_axis_name="s")
    sc_call = functools.partial(
        pl.kernel,
        out_type=jax.ShapeDtypeStruct((B * N * D,), jnp.float32),
        mesh=mesh,
        scratch_types=[
            pltpu.VMEM((chunk,), jnp.float32),
            pltpu.VMEM((chunk,), jnp.float32),
        ],
    )(functools.partial(_sc_add, B, N, D))
    out_flat = sc_call(patches_flat, table_flat)
    return out_flat.reshape(B, N, D)


# speedup vs baseline: 1.9574x; 1.9574x over previous
import functools

import jax
import jax.numpy as jnp
from jax import lax
from jax.experimental import pallas as pl
from jax.experimental.pallas import tpu as pltpu
from jax.experimental.pallas import tpu_sc as plsc

NUM_CORES = 2
NUM_SUBCORES = 16
NUM_WORKERS = NUM_CORES * NUM_SUBCORES
LANES = 16
NBUF = 8


def _sc_add(B, N, D, p_hbm, t_hbm, o_hbm, *refs):
    rows_per_worker = N // NUM_WORKERS
    chunk = rows_per_worker * D
    tbl_v = refs[0]
    bufs = refs[1 : 1 + NBUF]
    isems = refs[1 + NBUF : 1 + 2 * NBUF]
    osems = refs[1 + 2 * NBUF : 1 + 3 * NBUF]

    wid = lax.axis_index("s") * NUM_CORES + lax.axis_index("c")
    n0 = wid * chunk
    pltpu.sync_copy(t_hbm.at[pl.ds(n0, chunk)], tbl_v)

    def in_slice(b):
        return p_hbm.at[pl.ds(b * (N * D) + n0, chunk)]

    def out_slice(b):
        return o_hbm.at[pl.ds(b * (N * D) + n0, chunk)]

    def add(k):
        @plsc.parallel_loop(0, chunk, step=LANES, unroll=8)
        def _(i):
            s = pl.ds(i, LANES)
            plsc.addupdate(bufs[k].at[s], tbl_v[s])

    for k in range(NBUF // 2):
        pltpu.async_copy(in_slice(k), bufs[k], isems[k])

    def group(g, carry):
        for k in range(NBUF):
            b = g * NBUF + k
            pltpu.make_async_copy(in_slice(b), bufs[k], isems[k]).wait()
            add(k)
            ko = (k + NBUF // 2) % NBUF

            def drain_out():
                pltpu.make_async_copy(
                    bufs[ko], out_slice(b - NBUF // 2), osems[ko]
                ).wait()

            if k >= NBUF // 2:
                drain_out()
            else:
                pl.when(g > 0)(drain_out)
            pltpu.async_copy(bufs[k], out_slice(b), osems[k])

            def prefetch_in():
                pltpu.async_copy(in_slice(b + NBUF // 2), bufs[ko], isems[ko])

            if k < NBUF // 2:
                prefetch_in()
            else:
                pl.when(g < B // NBUF - 1)(prefetch_in)
        return carry

    lax.fori_loop(0, B // NBUF, group, 0)

    for k in range(NBUF // 2, NBUF):
        b = B - NBUF + k
        pltpu.make_async_copy(bufs[k], out_slice(b), osems[k]).wait()


def kernel(patches, pos_table):
    B, N, D = patches.shape
    table_flat = pos_table[:N].reshape(-1)
    patches_flat = patches.reshape(-1)
    chunk = (N // NUM_WORKERS) * D

    mesh = plsc.VectorSubcoreMesh(core_axis_name="c", subcore_axis_name="s")
    scratch = (
        [pltpu.VMEM((chunk,), jnp.float32)]
        + [pltpu.VMEM((chunk,), jnp.float32) for _ in range(NBUF)]
        + [pltpu.SemaphoreType.DMA for _ in range(2 * NBUF)]
    )
    sc_call = functools.partial(
        pl.kernel,
        out_type=jax.ShapeDtypeStruct((B * N * D,), jnp.float32),
        mesh=mesh,
        scratch_types=scratch,
    )(functools.partial(_sc_add, B, N, D))
    out_flat = sc_call(patches_flat, table_flat)
    return out_flat.reshape(B, N, D)


# speedup vs baseline: 5.7956x; 2.9609x over previous
import functools

import jax
import jax.numpy as jnp
from jax import lax
from jax.experimental import pallas as pl
from jax.experimental.pallas import tpu as pltpu
from jax.experimental.pallas import tpu_sc as plsc

NUM_CORES = 2
NUM_SUBCORES = 16
NUM_WORKERS = NUM_CORES * NUM_SUBCORES
LANES = 16
NBUF = 4
BATCH_GROUPS = 4
ROW_GROUPS = NUM_WORKERS // BATCH_GROUPS
CHUNKS = 3


def _sc_add(B, N, D, p_hbm, t_hbm, o_hbm, *refs):
    wrows = N // ROW_GROUPS
    crows = wrows // CHUNKS
    bpw = B // BATCH_GROUPS

    tbl_v = refs[0]
    bufs = refs[1 : 1 + NBUF]
    isems = refs[1 + NBUF : 1 + 2 * NBUF]
    osems = refs[1 + 2 * NBUF : 1 + 3 * NBUF]

    wid = lax.axis_index("s") * NUM_CORES + lax.axis_index("c")
    bg = wid // ROW_GROUPS
    rg = wid % ROW_GROUPS
    r0 = rg * wrows
    b0 = bg * bpw
    pltpu.sync_copy(t_hbm.at[pl.ds(r0, wrows)], tbl_v)

    def in_slice(b, c):
        return p_hbm.at[b0 + b, pl.ds(r0 + c * crows, crows)]

    def out_slice(b, c):
        return o_hbm.at[b0 + b, pl.ds(r0 + c * crows, crows)]

    def add(k, c):
        buf = bufs[k]

        @plsc.parallel_loop(0, crows)
        def _(r):
            tr = c * crows + r
            for v in range(D // LANES):
                s = pl.ds(v * LANES, LANES)
                plsc.addupdate(buf.at[r, s], tbl_v[tr, s])

    for c in range(CHUNKS):
        for k in range(NBUF // 2):
            pltpu.async_copy(in_slice(k, c), bufs[k], isems[k])

        def group(g, carry):
            for k in range(NBUF):
                b = g * NBUF + k
                pltpu.make_async_copy(in_slice(b, c), bufs[k], isems[k]).wait()
                add(k, c)
                ko = (k + NBUF // 2) % NBUF

                def drain_out():
                    pltpu.make_async_copy(
                        bufs[ko], out_slice(b - NBUF // 2, c), osems[ko]
                    ).wait()

                if k >= NBUF // 2:
                    drain_out()
                else:
                    pl.when(g > 0)(drain_out)
                pltpu.async_copy(bufs[k], out_slice(b, c), osems[k])

                def prefetch_in():
                    pltpu.async_copy(in_slice(b + NBUF // 2, c), bufs[ko], isems[ko])

                if k < NBUF // 2:
                    prefetch_in()
                else:
                    pl.when(g < bpw // NBUF - 1)(prefetch_in)
            return carry

        lax.fori_loop(0, bpw // NBUF, group, 0)

        for k in range(NBUF // 2, NBUF):
            b = bpw - NBUF + k
            pltpu.make_async_copy(bufs[k], out_slice(b, c), osems[k]).wait()


def kernel(patches, pos_table):
    B, N, D = patches.shape
    table = pos_table[:N]
    wrows = N // ROW_GROUPS
    crows = wrows // CHUNKS

    mesh = plsc.VectorSubcoreMesh(core_axis_name="c", subcore_axis_name="s")
    scratch = (
        [pltpu.VMEM((wrows, D), jnp.float32)]
        + [pltpu.VMEM((crows, D), jnp.float32) for _ in range(NBUF)]
        + [pltpu.SemaphoreType.DMA for _ in range(2 * NBUF)]
    )
    sc_call = functools.partial(
        pl.kernel,
        out_type=jax.ShapeDtypeStruct((B, N, D), jnp.float32),
        mesh=mesh,
        scratch_types=scratch,
    )(functools.partial(_sc_add, B, N, D))
    return sc_call(patches, table)
